# baseline (speedup 1.0000x reference)
import functools

import jax
import jax.numpy as jnp
from jax import lax
from jax.experimental import pallas as pl
from jax.experimental.pallas import tpu as pltpu
from jax.experimental.pallas import tpu_sc as plsc

B = 16384
K = 64
KH = 32
L = 16
NC = 2
NS = 16
NW = NC * NS
BPW = B // NW
CHUNK = 128
NCHUNK = BPW // CHUNK
NROW = 1000000
CB = 4096
NG = (NROW + 4 * CB - 1) // (4 * CB)
NTAB = NG * CB


def _tc_assemble_hi(tU, tI):
    def body(xu, xi, ou, oi):
        yu = jnp.transpose(xu[...])
        yi = jnp.transpose(xi[...])
        for q in range(4):
            ou[:, q * KH:(q + 1) * KH] = yu[q * CB:(q + 1) * CB, :]
            oi[:, q * KH:(q + 1) * KH] = yi[q * CB:(q + 1) * CB, :]

    tin = pl.BlockSpec((KH, 4 * CB), lambda i: (0, i))
    out = pl.BlockSpec((CB, 128), lambda i: (i, 0))
    return pl.pallas_call(
        body,
        grid=(NG,),
        in_specs=[tin, tin],
        out_specs=[out, out],
        out_shape=[jax.ShapeDtypeStruct((NTAB, 128), jnp.float32)] * 2,
    )(tU, tI)


def _sc_lo(sampleU, sampleI, sampleJ, betaI, gUlo, gIlo):
    mesh = plsc.VectorSubcoreMesh(core_axis_name="c", subcore_axis_name="s")

    @functools.partial(
        pl.kernel,
        out_type=jax.ShapeDtypeStruct((B,), jnp.float32),
        mesh=mesh,
        compiler_params=pltpu.CompilerParams(
            use_tc_tiling_on_sc=False, needs_layout_passes=False),
        scratch_types=[
            pltpu.VMEM((CHUNK,), jnp.int32),
            pltpu.VMEM((CHUNK,), jnp.int32),
            pltpu.VMEM((CHUNK,), jnp.int32),
            pltpu.VMEM((CHUNK, KH), jnp.float32),
            pltpu.VMEM((CHUNK, KH), jnp.float32),
            pltpu.VMEM((CHUNK, KH), jnp.float32),
            pltpu.VMEM((CHUNK,), jnp.float32),
            pltpu.VMEM((CHUNK,), jnp.float32),
            pltpu.VMEM((CHUNK,), jnp.float32),
            pltpu.SemaphoreType.DMA,
        ],
    )
    def k_lo(sU, sI, sJ, bI_hbm, gU_hbm, gI_hbm, diff_hbm,
             idxU, idxI, idxJ, gU, gI, gJ, bIv, bJv, dv, sem):
        wid = lax.axis_index("s") * NC + lax.axis_index("c")
        base = wid * BPW
        lanes = jnp.arange(L, dtype=jnp.int32)

        def chunk_body(ci, carry):
            cbase = base + ci * CHUNK
            pltpu.sync_copy(sU.at[pl.ds(cbase, CHUNK)], idxU)
            pltpu.sync_copy(sI.at[pl.ds(cbase, CHUNK)], idxI)
            pltpu.sync_copy(sJ.at[pl.ds(cbase, CHUNK)], idxJ)
            cp1 = pltpu.async_copy(gU_hbm.at[idxU], gU, sem)
            cp2 = pltpu.async_copy(gI_hbm.at[idxI], gI, sem)
            cp3 = pltpu.async_copy(gI_hbm.at[idxJ], gJ, sem)
            cp4 = pltpu.async_copy(bI_hbm.at[idxI], bIv, sem)
            cp5 = pltpu.async_copy(bI_hbm.at[idxJ], bJv, sem)
            cp1.wait()
            cp2.wait()
            cp3.wait()
            cp4.wait()
            cp5.wait()
            for g in range(CHUNK // L):
                sl = pl.ds(g * L, L)
                svec = jnp.full((L,), g * L, jnp.int32) + lanes
                acc = bIv[sl] - bJv[sl]
                for k in range(KH):
                    kv = lax.bitwise_and(lanes + k, KH - 1)
                    gu = plsc.load_gather(gU, [svec, kv])
                    gi = plsc.load_gather(gI, [svec, kv])
                    gj = plsc.load_gather(gJ, [svec, kv])
                    acc = acc + gu * (gi - gj)
                dv[sl] = acc
            pltpu.sync_copy(dv, diff_hbm.at[pl.ds(cbase, CHUNK)])
            return carry

        lax.fori_loop(0, NCHUNK, chunk_body, 0)

    return k_lo(sampleU, sampleI, sampleJ, betaI, gUlo, gIlo)


def _sc_hi(sampleU, sampleI, sampleJ, tabU, tabI):
    mesh = plsc.VectorSubcoreMesh(core_axis_name="c", subcore_axis_name="s")

    @functools.partial(
        pl.kernel,
        out_type=jax.ShapeDtypeStruct((B,), jnp.float32),
        mesh=mesh,
        compiler_params=pltpu.CompilerParams(
            use_tc_tiling_on_sc=True, needs_layout_passes=False),
        scratch_types=[
            pltpu.VMEM((CHUNK,), jnp.int32),
            pltpu.VMEM((CHUNK,), jnp.int32),
            pltpu.VMEM((CHUNK,), jnp.int32),
            pltpu.VMEM((CHUNK,), jnp.int32),
            pltpu.VMEM((CHUNK,), jnp.int32),
            pltpu.VMEM((CHUNK,), jnp.int32),
            pltpu.VMEM((CHUNK, 128), jnp.float32),
            pltpu.VMEM((CHUNK, 128), jnp.float32),
            pltpu.VMEM((CHUNK, 128), jnp.float32),
            pltpu.VMEM((CHUNK,), jnp.float32),
            pltpu.SemaphoreType.DMA,
        ],
    )
    def k_hi(sU, sI, sJ, tU_hbm, tI_hbm, diff_hbm,
             idxU, blkU, idxI, blkI, idxJ, blkJ, gU, gI, gJ, dv, sem):
        wid = lax.axis_index("s") * NC + lax.axis_index("c")
        base = wid * BPW
        lanes = jnp.arange(L, dtype=jnp.int32)

        def qsplit(v):
            blk = (lax.shift_left(lax.shift_right_logical(v, 14), 12)
                   + lax.bitwise_and(v, CB - 1))
            col = lax.shift_left(
                lax.bitwise_and(lax.shift_right_logical(v, 12), 3), 5)
            return blk, col

        def chunk_body(ci, carry):
            cbase = base + ci * CHUNK
            pltpu.sync_copy(sU.at[pl.ds(cbase, CHUNK)], idxU)
            pltpu.sync_copy(sI.at[pl.ds(cbase, CHUNK)], idxI)
            pltpu.sync_copy(sJ.at[pl.ds(cbase, CHUNK)], idxJ)
            for g in range(CHUNK // L):
                sl = pl.ds(g * L, L)
                blkU[sl], _ = qsplit(idxU[sl])
                blkI[sl], _ = qsplit(idxI[sl])
                blkJ[sl], _ = qsplit(idxJ[sl])
            cp1 = pltpu.async_copy(tU_hbm.at[blkU], gU, sem)
            cp2 = pltpu.async_copy(tI_hbm.at[blkI], gI, sem)
            cp3 = pltpu.async_copy(tI_hbm.at[blkJ], gJ, sem)
            cp1.wait()
            cp2.wait()
            cp3.wait()
            for g in range(CHUNK // L):
                sl = pl.ds(g * L, L)
                svec = jnp.full((L,), g * L, jnp.int32) + lanes
                _, cu = qsplit(idxU[sl])
                _, ci_ = qsplit(idxI[sl])
                _, cj = qsplit(idxJ[sl])
                acc = jnp.zeros((L,), jnp.float32)
                for k in range(KH):
                    kv = lax.bitwise_and(lanes + k, KH - 1)
                    gu = plsc.load_gather(gU, [svec, cu + kv])
                    gi = plsc.load_gather(gI, [svec, ci_ + kv])
                    gj = plsc.load_gather(gJ, [svec, cj + kv])
                    acc = acc + gu * (gi - gj)
                dv[sl] = acc
            pltpu.sync_copy(dv, diff_hbm.at[pl.ds(cbase, CHUNK)])
            return carry

        lax.fori_loop(0, NCHUNK, chunk_body, 0)

    return k_hi(sampleU, sampleI, sampleJ, tabU, tabI)


def _tc_loss(d_lo, d_hi):
    def body(a_ref, b_ref, out_ref):
        x = a_ref[...] + b_ref[...]
        loss = -jnp.mean(jnp.log(jax.nn.sigmoid(x)))
        out_ref[...] = loss.reshape(1, 1)

    out = pl.pallas_call(
        body,
        out_shape=jax.ShapeDtypeStruct((1, 1), jnp.float32),
    )(d_lo.reshape(B // 128, 128), d_hi.reshape(B // 128, 128))
    return out[0, 0]


def kernel(sampleU, sampleI, sampleJ, betaI, gammaU, gammaI):
    gUlo = lax.slice(gammaU, (0, 0), (NROW, KH))
    gIlo = lax.slice(gammaI, (0, 0), (NROW, KH))
    tU = lax.slice(gammaU, (0, KH), (NROW, K)).T
    tI = lax.slice(gammaI, (0, KH), (NROW, K)).T
    tabU, tabI = _tc_assemble_hi(tU, tI)
    d_lo = _sc_lo(sampleU, sampleI, sampleJ, betaI, gUlo, gIlo)
    d_hi = _sc_hi(sampleU, sampleI, sampleJ, tabU, tabI)
    return _tc_loss(d_lo, d_hi)


# speedup vs baseline: 3.8800x; 3.8800x over previous
import functools

import jax
import jax.numpy as jnp
from jax import lax
from jax.experimental import pallas as pl
from jax.experimental.pallas import tpu as pltpu
from jax.experimental.pallas import tpu_sc as plsc

B = 16384
K = 64
L = 16
NC = 2
NS = 16
NW = NC * NS
BPW = B // NW
CHUNK = 64
NCHUNK = BPW // CHUNK


def _sc_diffs(sampleU, sampleI, sampleJ, betaI, gammaU, gammaI):
    mesh = plsc.VectorSubcoreMesh(core_axis_name="c", subcore_axis_name="s")

    @functools.partial(
        pl.kernel,
        out_type=jax.ShapeDtypeStruct((B,), jnp.float32),
        mesh=mesh,
        compiler_params=pltpu.CompilerParams(
            use_tc_tiling_on_sc=True, needs_layout_passes=False),
        scratch_types=[
            pltpu.VMEM((CHUNK,), jnp.int32),
            pltpu.VMEM((CHUNK,), jnp.int32),
            pltpu.VMEM((CHUNK,), jnp.int32),
            pltpu.VMEM((CHUNK, K), jnp.float32),
            pltpu.VMEM((CHUNK, K), jnp.float32),
            pltpu.VMEM((CHUNK, K), jnp.float32),
            pltpu.VMEM((CHUNK,), jnp.float32),
            pltpu.VMEM((CHUNK,), jnp.float32),
            pltpu.VMEM((CHUNK,), jnp.float32),
            pltpu.SemaphoreType.DMA,
            pltpu.SemaphoreType.DMA,
        ],
    )
    def sc_kernel(sU, sI, sJ, bI_hbm, gU_hbm, gI_hbm, diff_hbm,
                  idxU, idxI, idxJ, gU, gI, gJ, bIv, bJv, dv, sem, sem2):
        wid = lax.axis_index("s") * NC + lax.axis_index("c")
        base = wid * BPW
        lanes = jnp.arange(L, dtype=jnp.int32)

        def chunk_body(ci, carry):
            cbase = base + ci * CHUNK
            pltpu.sync_copy(sU.at[pl.ds(cbase, CHUNK)], idxU)
            pltpu.sync_copy(sI.at[pl.ds(cbase, CHUNK)], idxI)
            pltpu.sync_copy(sJ.at[pl.ds(cbase, CHUNK)], idxJ)
            cp4 = pltpu.async_copy(bI_hbm.at[idxI], bIv, sem2)
            cp5 = pltpu.async_copy(bI_hbm.at[idxJ], bJv, sem2)

            for g in range(CHUNK // L):
                sl = pl.ds(g * L, L)
                vu = idxU[sl]
                vi = idxI[sl]
                vj = idxJ[sl]
                for l in range(L):
                    s = g * L + l
                    u = vu[l]
                    i = vi[l]
                    j = vj[l]
                    pltpu.async_copy(gU_hbm.at[u], gU.at[s], sem)
                    pltpu.async_copy(
                        gI_hbm.at[lax.shift_right_logical(i, 3),
                                  lax.bitwise_and(i, 7)], gI.at[s], sem)
                    pltpu.async_copy(
                        gI_hbm.at[lax.shift_right_logical(j, 3),
                                  lax.bitwise_and(j, 7)], gJ.at[s], sem)
            for s in range(CHUNK):
                pltpu.make_async_copy(gU_hbm.at[0], gU.at[s], sem).wait()
                pltpu.make_async_copy(gU_hbm.at[0], gI.at[s], sem).wait()
                pltpu.make_async_copy(gU_hbm.at[0], gJ.at[s], sem).wait()
            cp4.wait()
            cp5.wait()

            for g in range(CHUNK // L):
                sl = pl.ds(g * L, L)
                svec = jnp.full((L,), g * L, jnp.int32) + lanes
                acc = bIv[sl] - bJv[sl]
                for k in range(K):
                    kv = lax.bitwise_and(lanes + k, K - 1)
                    gu = plsc.load_gather(gU, [svec, kv])
                    gi = plsc.load_gather(gI, [svec, kv])
                    gj = plsc.load_gather(gJ, [svec, kv])
                    acc = acc + gu * (gi - gj)
                dv[sl] = acc

            pltpu.sync_copy(dv, diff_hbm.at[pl.ds(cbase, CHUNK)])
            return carry

        lax.fori_loop(0, NCHUNK, chunk_body, 0)

    return sc_kernel(sampleU, sampleI, sampleJ, betaI, gammaU, gammaI)


def _tc_loss(diffs):
    def body(d_ref, out_ref):
        loss = -jnp.mean(jnp.log(jax.nn.sigmoid(d_ref[...])))
        out_ref[...] = loss.reshape(1, 1)

    out = pl.pallas_call(
        body,
        out_shape=jax.ShapeDtypeStruct((1, 1), jnp.float32),
    )(diffs.reshape(B // 128, 128))
    return out[0, 0]


def kernel(sampleU, sampleI, sampleJ, betaI, gammaU, gammaI):
    gI3 = gammaI.reshape(1000000 // 8, 8, K)
    diffs = _sc_diffs(sampleU, sampleI, sampleJ, betaI, gammaU, gI3)
    return _tc_loss(diffs)


# speedup vs baseline: 4.1672x; 1.0740x over previous
import functools

import jax
import jax.numpy as jnp
from jax import lax
from jax.experimental import pallas as pl
from jax.experimental.pallas import tpu as pltpu
from jax.experimental.pallas import tpu_sc as plsc

B = 16384
K = 64
L = 16
NC = 2
NS = 16
NW = NC * NS
BPW = B // NW
CHUNK = 64
NCHUNK = BPW // CHUNK


def _sc_diffs(sampleU, sampleI, sampleJ, betaI, gammaU, gammaI):
    mesh = plsc.VectorSubcoreMesh(core_axis_name="c", subcore_axis_name="s")

    @functools.partial(
        pl.kernel,
        out_type=jax.ShapeDtypeStruct((B,), jnp.float32),
        mesh=mesh,
        compiler_params=pltpu.CompilerParams(
            use_tc_tiling_on_sc=True, needs_layout_passes=False),
        scratch_types=[
            pltpu.VMEM((CHUNK,), jnp.int32),
            pltpu.VMEM((CHUNK,), jnp.int32),
            pltpu.VMEM((CHUNK,), jnp.int32),
            pltpu.VMEM((CHUNK, K), jnp.float32),
            pltpu.VMEM((CHUNK, K), jnp.float32),
            pltpu.VMEM((CHUNK, K), jnp.float32),
            pltpu.VMEM((CHUNK,), jnp.float32),
            pltpu.VMEM((CHUNK,), jnp.float32),
            pltpu.VMEM((CHUNK,), jnp.float32),
            pltpu.SemaphoreType.DMA,
            pltpu.SemaphoreType.DMA,
        ],
    )
    def sc_kernel(sU, sI, sJ, bI_hbm, gU_hbm, gI_hbm, diff_hbm,
                  idxU, idxI, idxJ, gU, gI, gJ, bIv, bJv, dv, sem, sem2):
        wid = lax.axis_index("s") * NC + lax.axis_index("c")
        base = wid * BPW
        lanes = jnp.arange(L, dtype=jnp.int32)

        def chunk_body(ci, carry):
            cbase = base + ci * CHUNK
            pltpu.sync_copy(sU.at[pl.ds(cbase, CHUNK)], idxU)
            pltpu.sync_copy(sI.at[pl.ds(cbase, CHUNK)], idxI)
            pltpu.sync_copy(sJ.at[pl.ds(cbase, CHUNK)], idxJ)
            cp4 = pltpu.async_copy(bI_hbm.at[idxI], bIv, sem2)
            cp5 = pltpu.async_copy(bI_hbm.at[idxJ], bJv, sem2)

            for g in range(CHUNK // L):
                sl = pl.ds(g * L, L)
                vu = idxU[sl]
                vi = idxI[sl]
                vj = idxJ[sl]
                for l in range(L):
                    s = g * L + l
                    u = vu[l]
                    i = vi[l]
                    j = vj[l]
                    pltpu.async_copy(
                        gU_hbm.at[lax.shift_right_logical(u, 3),
                                  lax.bitwise_and(u, 7)], gU.at[s], sem)
                    pltpu.async_copy(
                        gI_hbm.at[lax.shift_right_logical(i, 3),
                                  lax.bitwise_and(i, 7)], gI.at[s], sem)
                    pltpu.async_copy(
                        gI_hbm.at[lax.shift_right_logical(j, 3),
                                  lax.bitwise_and(j, 7)], gJ.at[s], sem)
            for s in range(CHUNK):
                pltpu.make_async_copy(gU_hbm.at[0, 0], gU.at[s], sem).wait()
                pltpu.make_async_copy(gU_hbm.at[0, 0], gI.at[s], sem).wait()
                pltpu.make_async_copy(gU_hbm.at[0, 0], gJ.at[s], sem).wait()
            cp4.wait()
            cp5.wait()

            for g in range(CHUNK // L):
                sl = pl.ds(g * L, L)
                svec = jnp.full((L,), g * L, jnp.int32) + lanes
                acc = bIv[sl] - bJv[sl]
                for k in range(K):
                    kv = lax.bitwise_and(lanes + k, K - 1)
                    gu = plsc.load_gather(gU, [svec, kv])
                    gi = plsc.load_gather(gI, [svec, kv])
                    gj = plsc.load_gather(gJ, [svec, kv])
                    acc = acc + gu * (gi - gj)
                dv[sl] = acc

            pltpu.sync_copy(dv, diff_hbm.at[pl.ds(cbase, CHUNK)])
            return carry

        lax.fori_loop(0, NCHUNK, chunk_body, 0)

    return sc_kernel(sampleU, sampleI, sampleJ, betaI, gammaU, gammaI)


def _tc_loss(diffs):
    def body(d_ref, out_ref):
        loss = -jnp.mean(jnp.log(jax.nn.sigmoid(d_ref[...])))
        out_ref[...] = loss.reshape(1, 1)

    out = pl.pallas_call(
        body,
        out_shape=jax.ShapeDtypeStruct((1, 1), jnp.float32),
    )(diffs.reshape(B // 128, 128))
    return out[0, 0]


def kernel(sampleU, sampleI, sampleJ, betaI, gammaU, gammaI):
    gU3 = gammaU.reshape(1000000 // 8, 8, K)
    gI3 = gammaI.reshape(1000000 // 8, 8, K)
    diffs = _sc_diffs(sampleU, sampleI, sampleJ, betaI, gU3, gI3)
    return _tc_loss(diffs)
